# baseline (speedup 1.0000x reference)
import functools

import jax
import jax.numpy as jnp
from jax import lax
from jax.experimental import pallas as pl
from jax.experimental.pallas import tpu as pltpu
from jax.experimental.pallas import tpu_sc as plsc

B = 16384
N_COND = 4
SUB = 32
D = SUB * N_COND
CD = 512
VOCAB = 1000000

NC, NS = 2, 16
NW = NC * NS
BPW = B // NW

GR = 2
NSLOT = N_COND * GR
NG = BPW // GR
HROWS = 128


@functools.cache
def _build_gather_sc():
    mesh = plsc.VectorSubcoreMesh(core_axis_name="c", subcore_axis_name="s")

    @functools.partial(
        pl.kernel,
        mesh=mesh,
        compiler_params=pltpu.CompilerParams(needs_layout_passes=False),
        out_type=jax.ShapeDtypeStruct((B, D), jnp.float32),
        scratch_types=[
            pltpu.VMEM((BPW + 16,), jnp.int32),
            pltpu.VMEM((BPW + 16,), jnp.int32),
            pltpu.VMEM((BPW + 16,), jnp.int32),
            pltpu.VMEM((BPW + 16,), jnp.int32),
            pltpu.VMEM((NSLOT, SUB, 128), jnp.float32),
            pltpu.VMEM((NSLOT, SUB, 128), jnp.float32),
            pltpu.VMEM((NSLOT, SUB, 128), jnp.float32),
            pltpu.VMEM((HROWS, D), jnp.float32),
            pltpu.SemaphoreType.DMA,
            pltpu.SemaphoreType.DMA,
            pltpu.SemaphoreType.DMA,
        ],
    )
    def _gather_sc(i0, i1, i2, i3, e0, e1, e2, e3, out,
                   v0, v1, v2, v3, ring0, ring1, ring2, h,
                   sem0, sem1, sem2):
        wid = lax.axis_index("s") * NC + lax.axis_index("c")
        base = wid * BPW
        idx_v = (v0, v1, v2, v3)
        tables = (e0, e1, e2, e3)
        rings = (ring0, ring1, ring2)
        sems = (sem0, sem1, sem2)
        idx_hbm = (i0, i1, i2, i3)
        lanes = lax.iota(jnp.int32, 16)
        for f in range(N_COND):
            pltpu.sync_copy(idx_hbm[f].at[pl.ds(base, BPW)],
                            idx_v[f].at[pl.ds(0, BPW)])

        def issue(g, p):
            for f in range(N_COND):
                vec = idx_v[f][pl.ds(g * GR, 16)]
                for bb in range(GR):
                    idx = vec[bb]
                    col0 = pl.multiple_of(jnp.bitwise_and(idx, -128), 128)
                    pltpu.async_copy(
                        tables[f].at[:, pl.ds(col0, 128)],
                        rings[p].at[f * GR + bb], sems[p])

        def drain(p):
            for _ in range(NSLOT):
                pltpu.make_async_copy(
                    tables[0].at[:, pl.ds(0, 128)], rings[p].at[0],
                    sems[p]).wait()

        def extract(g, p):
            for f in range(N_COND):
                vec = idx_v[f][pl.ds(g * GR, 16)]
                for bb in range(GR):
                    r = g * GR + bb
                    lane = jnp.full((16,), jnp.bitwise_and(vec[bb], 127),
                                    jnp.int32)
                    slot = f * GR + bb
                    lo = plsc.load_gather(rings[p].at[slot], [lanes, lane])
                    hi = plsc.load_gather(rings[p].at[slot],
                                          [lanes + 16, lane])
                    hr = jnp.bitwise_and(r, HROWS - 1)
                    h[hr, pl.ds(f * SUB, 16)] = lo
                    h[hr, pl.ds(f * SUB + 16, 16)] = hi

        def flush_after(g):
            @pl.when(jnp.bitwise_and(g, HROWS // GR - 1) == HROWS // GR - 1)
            def _():
                row0 = pl.multiple_of(base + (GR * (g + 1) - HROWS), HROWS)
                pltpu.sync_copy(h, out.at[pl.ds(row0, HROWS), :])

        def step(g, p):
            drain(p)
            extract(g, p)
            flush_after(g)

            @pl.when(g + 3 < NG)
            def _():
                issue(g + 3, p)

        issue(0, 0)
        issue(1, 1)
        issue(2, 2)

        def tri_body(i, carry):
            g = 3 * i
            step(g, 0)
            step(g + 1, 1)
            step(g + 2, 2)
            return carry

        lax.fori_loop(0, NG // 3, tri_body, 0)
        drain(0)
        extract(NG - 1, 0)
        flush_after(NG - 1)

    return _gather_sc


def _mlp_body(h_ref, w_ref, b_ref, o_ref):
    y = jnp.dot(h_ref[...], w_ref[...], preferred_element_type=jnp.float32)
    y = y + b_ref[...]
    o_ref[...] = y * jax.nn.sigmoid(y)


_MLP_BLK = 4096

_mlp = pl.pallas_call(
    _mlp_body,
    grid=(B // _MLP_BLK,),
    in_specs=[
        pl.BlockSpec((_MLP_BLK, D), lambda i: (i, 0)),
        pl.BlockSpec((D, CD), lambda i: (0, 0)),
        pl.BlockSpec((1, CD), lambda i: (0, 0)),
    ],
    out_specs=pl.BlockSpec((_MLP_BLK, CD), lambda i: (i, 0)),
    out_shape=jax.ShapeDtypeStruct((B, CD), jnp.float32),
)


def kernel(cond, E0, E1, E2, E3, W, b):
    cond = cond.astype(jnp.int32)
    h = _build_gather_sc()(
        cond[:, 0], cond[:, 1], cond[:, 2], cond[:, 3],
        E0.T, E1.T, E2.T, E3.T)
    return _mlp(h, W, b.reshape(1, CD))


# speedup vs baseline: 1.0055x; 1.0055x over previous
import functools

import jax
import jax.numpy as jnp
from jax import lax
from jax.experimental import pallas as pl
from jax.experimental.pallas import tpu as pltpu
from jax.experimental.pallas import tpu_sc as plsc

B = 16384
N_COND = 4
SUB = 32
D = SUB * N_COND
CD = 512
VOCAB = 1000000

NC, NS = 2, 16
NW = NC * NS
BPW = B // NW

GR = 2
NSLOT = N_COND * GR
NG = BPW // GR
HROWS = 128


@functools.cache
def _build_gather_sc():
    mesh = plsc.VectorSubcoreMesh(core_axis_name="c", subcore_axis_name="s")

    @functools.partial(
        pl.kernel,
        mesh=mesh,
        compiler_params=pltpu.CompilerParams(needs_layout_passes=False),
        out_type=jax.ShapeDtypeStruct((B, D), jnp.float32),
        scratch_types=[
            pltpu.VMEM((BPW + 16,), jnp.int32),
            pltpu.VMEM((BPW + 16,), jnp.int32),
            pltpu.VMEM((BPW + 16,), jnp.int32),
            pltpu.VMEM((BPW + 16,), jnp.int32),
            pltpu.VMEM((NSLOT, SUB, 128), jnp.float32),
            pltpu.VMEM((NSLOT, SUB, 128), jnp.float32),
            pltpu.VMEM((NSLOT, SUB, 128), jnp.float32),
            pltpu.VMEM((HROWS, D), jnp.float32),
            pltpu.SemaphoreType.DMA,
            pltpu.SemaphoreType.DMA,
            pltpu.SemaphoreType.DMA,
        ],
    )
    def _gather_sc(i0, i1, i2, i3, e0, e1, e2, e3, out,
                   v0, v1, v2, v3, ring0, ring1, ring2, h,
                   sem0, sem1, sem2):
        wid = lax.axis_index("s") * NC + lax.axis_index("c")
        base = wid * BPW
        idx_v = (v0, v1, v2, v3)
        tables = (e0, e1, e2, e3)
        rings = (ring0, ring1, ring2)
        sems = (sem0, sem1, sem2)
        idx_hbm = (i0, i1, i2, i3)
        lanes = lax.iota(jnp.int32, 16)
        idx_cps = [
            pltpu.async_copy(idx_hbm[f].at[pl.ds(base, BPW)],
                             idx_v[f].at[pl.ds(0, BPW)], sem0)
            for f in range(N_COND)
        ]
        for cp in idx_cps:
            cp.wait()

        def issue(g, p):
            for f in range(N_COND):
                vec = idx_v[f][pl.ds(g * GR, 16)]
                for bb in range(GR):
                    idx = vec[bb]
                    col0 = pl.multiple_of(jnp.bitwise_and(idx, -128), 128)
                    pltpu.async_copy(
                        tables[f].at[:, pl.ds(col0, 128)],
                        rings[p].at[f * GR + bb], sems[p])

        def drain(p):
            for _ in range(NSLOT):
                pltpu.make_async_copy(
                    tables[0].at[:, pl.ds(0, 128)], rings[p].at[0],
                    sems[p]).wait()

        def extract(g, p):
            for f in range(N_COND):
                vec = idx_v[f][pl.ds(g * GR, 16)]
                for bb in range(GR):
                    r = g * GR + bb
                    lane = jnp.full((16,), jnp.bitwise_and(vec[bb], 127),
                                    jnp.int32)
                    slot = f * GR + bb
                    lo = plsc.load_gather(rings[p].at[slot], [lanes, lane])
                    hi = plsc.load_gather(rings[p].at[slot],
                                          [lanes + 16, lane])
                    hr = jnp.bitwise_and(r, HROWS - 1)
                    h[hr, pl.ds(f * SUB, 16)] = lo
                    h[hr, pl.ds(f * SUB + 16, 16)] = hi

        def flush_after(g):
            @pl.when(jnp.bitwise_and(g, HROWS // GR - 1) == HROWS // GR - 1)
            def _():
                row0 = pl.multiple_of(base + (GR * (g + 1) - HROWS), HROWS)
                pltpu.sync_copy(h, out.at[pl.ds(row0, HROWS), :])

        def step(g, p):
            drain(p)
            extract(g, p)
            flush_after(g)

            @pl.when(g + 3 < NG)
            def _():
                issue(g + 3, p)

        issue(0, 0)
        issue(1, 1)
        issue(2, 2)

        def tri_body(i, carry):
            g = 3 * i
            step(g, 0)
            step(g + 1, 1)
            step(g + 2, 2)
            return carry

        lax.fori_loop(0, NG // 3, tri_body, 0)
        drain(0)
        extract(NG - 1, 0)
        flush_after(NG - 1)

    return _gather_sc


def _mlp_body(h_ref, w_ref, b_ref, o_ref):
    y = jnp.dot(h_ref[...], w_ref[...], preferred_element_type=jnp.float32)
    y = y + b_ref[...]
    o_ref[...] = y * jax.nn.sigmoid(y)


_MLP_BLK = 4096

_mlp = pl.pallas_call(
    _mlp_body,
    grid=(B // _MLP_BLK,),
    in_specs=[
        pl.BlockSpec((_MLP_BLK, D), lambda i: (i, 0)),
        pl.BlockSpec((D, CD), lambda i: (0, 0)),
        pl.BlockSpec((1, CD), lambda i: (0, 0)),
    ],
    out_specs=pl.BlockSpec((_MLP_BLK, CD), lambda i: (i, 0)),
    out_shape=jax.ShapeDtypeStruct((B, CD), jnp.float32),
)


def kernel(cond, E0, E1, E2, E3, W, b):
    cond = cond.astype(jnp.int32)
    h = _build_gather_sc()(
        cond[:, 0], cond[:, 1], cond[:, 2], cond[:, 3],
        E0.T, E1.T, E2.T, E3.T)
    return _mlp(h, W, b.reshape(1, CD))
